# baseline (speedup 1.0000x reference)
import jax
import jax.numpy as jnp
from jax import lax
from jax.experimental import pallas as pl
from jax.experimental.pallas import tpu as pltpu
from jax.experimental.pallas import tpu_sc as plsc

N = 10000
D = 128
E = 320000
NPAD = N + 16
NC = 2
NS = 16
NW = NC * NS
K = 120
C = 84
EPAD = NW * C * K
NACC = 10112
RPT = NACC // NS

_PIECES = [(k * K, K) for k in range(RPT // K)]
if RPT % K:
    _PIECES.append((RPT - RPT % K, RPT % K))

_MESH = dict(core_axis_name="c", subcore_axis_name="s",
             num_cores=NC, num_subcores=NS)


def _worker():
    c = lax.axis_index("c")
    s = lax.axis_index("s")
    wid = c * NS + s
    return c, s * RPT, wid * C * K


def _make_agg_kernel(with_deg: bool = False):
    scratch = [
        pltpu.VMEM_SHARED((NACC, D), jnp.float32),
        pltpu.VMEM((K, D), jnp.float32),
        pltpu.VMEM((K, D), jnp.float32),
        pltpu.SemaphoreType.DMA,
        pltpu.SemaphoreType.DMA,
    ] + [pltpu.VMEM((K,), jnp.int32) for _ in range(8)] \
      + [pltpu.SemaphoreType.DMA for _ in range(4)]

    def body(feat, srcc, dstc, *rest):
        if with_deg:
            (out, outd, acc, r0, r1, sem0, sem1,
             si0, di0, si1, di1, si2, di2, si3, di3,
             smi0, smi1, smi2, smi3) = rest
        else:
            (out, acc, r0, r1, sem0, sem1,
             si0, di0, si1, di1, si2, di2, si3, di3,
             smi0, smi1, smi2, smi3) = rest
            outd = None
        c, rowbase, base = _worker()
        rows = (r0, r1)
        gsem = (sem0, sem1)
        sidx = (si0, si1, si2, si3)
        didx = (di0, di1, di2, di3)
        isem = (smi0, smi1, smi2, smi3)

        def idx_start(jc, p):
            pltpu.async_copy(srcc.at[pl.ds(base + jc * K, K)], sidx[p],
                             isem[p])
            pltpu.async_copy(dstc.at[pl.ds(base + jc * K, K)], didx[p],
                             isem[p])

        def idx_wait(jc, p):
            pltpu.make_async_copy(srcc.at[pl.ds(base + jc * K, K)], sidx[p],
                                  isem[p]).wait()
            pltpu.make_async_copy(dstc.at[pl.ds(base + jc * K, K)], didx[p],
                                  isem[p]).wait()

        for p in range(4):
            idx_start(p, p)

        def fill_body(i, carry):
            zero = jnp.zeros((16,), jnp.float32)
            for l in range(D // 16):
                r0[i, pl.ds(l * 16, 16)] = zero
            return carry

        lax.fori_loop(0, K, fill_body, 0)
        for off, sz in _PIECES:
            pltpu.sync_copy(r0.at[pl.ds(0, sz)],
                            acc.at[pl.ds(rowbase + off, sz)])
        idx_wait(0, 0)
        pltpu.async_copy(feat.at[sidx[0]], r0, sem0)
        idx_wait(1, 1)
        pltpu.async_copy(feat.at[sidx[1]], r1, sem1)
        plsc.subcore_barrier()

        def step(jc, b, p, prefetch_idx, start_next):
            pltpu.make_async_copy(feat.at[sidx[p]], rows[b], gsem[b]).wait()
            pltpu.sync_copy(rows[b], acc.at[didx[p]], add=True)
            if prefetch_idx:
                idx_start(jc + 4, p)
            if start_next:
                pn = (p + 2) % 4
                idx_wait(jc + 2, pn)
                pltpu.async_copy(feat.at[sidx[pn]], rows[b], gsem[b])

        def loop_body(j4, carry):
            for b4 in range(4):
                step(4 * j4 + b4, b4 % 2, b4, True, True)
            return carry

        lax.fori_loop(0, (C - 4) // 4, loop_body, 0)
        for b4 in range(4):
            step(C - 4 + b4, b4 % 2, b4, False, b4 < 2)

        plsc.subcore_barrier()
        for off, sz in _PIECES:
            pltpu.sync_copy(acc.at[pl.ds(rowbase + off, sz)],
                            r0.at[pl.ds(0, sz)])
            pltpu.sync_copy(r0.at[pl.ds(0, sz)],
                            out.at[c, pl.ds(rowbase + off, sz)])

        if not with_deg:
            return

        def fill2_body(i, carry):
            zero = jnp.zeros((16,), jnp.float32)
            one = jnp.full((16,), 1.0, dtype=jnp.float32)
            for l in range(D // 16):
                r0[i, pl.ds(l * 16, 16)] = zero
                r1[i, pl.ds(l * 16, 16)] = one
            return carry

        lax.fori_loop(0, K, fill2_body, 0)

        def didx_start(jc, p):
            pltpu.async_copy(dstc.at[pl.ds(base + jc * K, K)], didx[p],
                             isem[p])

        def didx_wait(jc, p):
            pltpu.make_async_copy(dstc.at[pl.ds(base + jc * K, K)], didx[p],
                                  isem[p]).wait()

        didx_start(0, 0)
        didx_start(1, 1)
        for off, sz in _PIECES:
            pltpu.sync_copy(r0.at[pl.ds(0, sz)],
                            acc.at[pl.ds(rowbase + off, sz)])
        plsc.subcore_barrier()

        def dstep(jc, p, prefetch_idx):
            didx_wait(jc, p)
            pltpu.sync_copy(r1, acc.at[didx[p]], add=True)
            if prefetch_idx:
                didx_start(jc + 2, p)

        def dloop_body(j2, carry):
            for b2 in range(2):
                dstep(2 * j2 + b2, b2, True)
            return carry

        lax.fori_loop(0, (C - 2) // 2, dloop_body, 0)
        dstep(C - 2, 0, False)
        dstep(C - 1, 1, False)

        plsc.subcore_barrier()
        for off, sz in _PIECES:
            pltpu.sync_copy(acc.at[pl.ds(rowbase + off, sz)],
                            r0.at[pl.ds(0, sz)])
            pltpu.sync_copy(r0.at[pl.ds(0, sz)],
                            outd.at[c, pl.ds(rowbase + off, sz)])

    out_type = jax.ShapeDtypeStruct((NC, NACC, D), jnp.float32)
    return pl.kernel(
        body,
        out_type=(out_type, out_type) if with_deg else out_type,
        mesh=plsc.VectorSubcoreMesh(**_MESH), scratch_types=scratch,
        name="sage_agg_deg" if with_deg else "sage_agg")


_R = 400


def _make_lin_kernel(relu: bool):
    def body(p0, p1, d0, d1, xb, wl, wr, bb, ob):
        deg = jnp.maximum(d0[:, 0:1] + d1[:, 0:1], 1.0)
        mean = (p0[...] + p1[...]) / deg
        acc = jnp.dot(mean, wl[...], preferred_element_type=jnp.float32)
        acc = acc + jnp.dot(xb[...], wr[...],
                            preferred_element_type=jnp.float32)
        acc = acc + bb[...]
        if relu:
            acc = jnp.maximum(acc, 0.0)
        ob[...] = acc

    row = lambda i: (i, 0)
    full = lambda i: (0, 0)
    return pl.pallas_call(
        body,
        grid=(N // _R,),
        in_specs=[
            pl.BlockSpec((_R, D), row),
            pl.BlockSpec((_R, D), row),
            pl.BlockSpec((_R, D), row),
            pl.BlockSpec((_R, D), row),
            pl.BlockSpec((_R, D), row),
            pl.BlockSpec((D, D), full),
            pl.BlockSpec((D, D), full),
            pl.BlockSpec((1, D), full),
        ],
        out_specs=pl.BlockSpec((_R, D), row),
        out_shape=jax.ShapeDtypeStruct((N, D), jnp.float32),
        name="sage_lin_relu" if relu else "sage_lin",
    )


_agg = _make_agg_kernel()
_agg_deg = _make_agg_kernel(with_deg=True)
_lin_relu = _make_lin_kernel(relu=True)
_lin = _make_lin_kernel(relu=False)


def kernel(x, edge_index, W_l1, W_r1, b1, W_l2, W_r2, b2):
    src = edge_index[0].astype(jnp.int32)
    dst = edge_index[1].astype(jnp.int32)
    npd = EPAD - E
    pad_src = N + (jnp.arange(npd, dtype=jnp.int32) % 16)
    pad_dst = N + (jnp.arange(npd, dtype=jnp.int32) % (NACC - N))
    srcc = jnp.concatenate([src, pad_src])
    dstc = jnp.concatenate([dst, pad_dst])

    x_pad = jnp.pad(x, ((0, NPAD - N), (0, 0)))

    part1, degp = _agg_deg(x_pad, srcc, dstc)
    h = _lin_relu(part1[0], part1[1], degp[0], degp[1], x,
                  W_l1, W_r1, b1.reshape(1, D))
    h_pad = jnp.pad(h, ((0, NPAD - N), (0, 0)))
    part2 = _agg(h_pad, srcc, dstc)
    out = _lin(part2[0], part2[1], degp[0], degp[1], h,
               W_l2, W_r2, b2.reshape(1, D))
    return out


# speedup vs baseline: 1.1270x; 1.1270x over previous
import jax
import jax.numpy as jnp
from jax import lax
from jax.experimental import pallas as pl
from jax.experimental.pallas import tpu as pltpu
from jax.experimental.pallas import tpu_sc as plsc

N = 10000
D = 128
E = 320000
NPAD = N + 16
NC = 2
NS = 16
NW = NC * NS
K = 120
C = 84
EPAD = NW * C * K
NACC = 10112
RPT = NACC // NS

_PIECES = [(k * K, K) for k in range(RPT // K)]
if RPT % K:
    _PIECES.append((RPT - RPT % K, RPT % K))

_MESH = dict(core_axis_name="c", subcore_axis_name="s",
             num_cores=NC, num_subcores=NS)


def _worker():
    c = lax.axis_index("c")
    s = lax.axis_index("s")
    wid = c * NS + s
    return c, s * RPT, wid * C * K


def _make_agg_kernel(with_deg: bool = False):
    scratch = [
        pltpu.VMEM_SHARED((NACC, D), jnp.float32),
        pltpu.VMEM((K, D), jnp.float32),
        pltpu.VMEM((K, D), jnp.float32),
        pltpu.SemaphoreType.DMA,
        pltpu.SemaphoreType.DMA,
    ] + [pltpu.VMEM((K,), jnp.int32) for _ in range(8)] \
      + [pltpu.SemaphoreType.DMA for _ in range(4)]

    def body(feat, srcc, dstc, *rest):
        if with_deg:
            (out, outd, acc, r0, r1, sem0, sem1,
             si0, di0, si1, di1, si2, di2, si3, di3,
             smi0, smi1, smi2, smi3) = rest
        else:
            (out, acc, r0, r1, sem0, sem1,
             si0, di0, si1, di1, si2, di2, si3, di3,
             smi0, smi1, smi2, smi3) = rest
            outd = None
        c, rowbase, base = _worker()
        rows = (r0, r1)
        gsem = (sem0, sem1)
        sidx = (si0, si1, si2, si3)
        didx = (di0, di1, di2, di3)
        isem = (smi0, smi1, smi2, smi3)

        def idx_start(jc, p):
            pltpu.async_copy(srcc.at[pl.ds(base + jc * K, K)], sidx[p],
                             isem[p])
            pltpu.async_copy(dstc.at[pl.ds(base + jc * K, K)], didx[p],
                             isem[p])

        def idx_wait(jc, p):
            pltpu.make_async_copy(srcc.at[pl.ds(base + jc * K, K)], sidx[p],
                                  isem[p]).wait()
            pltpu.make_async_copy(dstc.at[pl.ds(base + jc * K, K)], didx[p],
                                  isem[p]).wait()

        for p in range(4):
            idx_start(p, p)

        def fill_body(i, carry):
            zero = jnp.zeros((16,), jnp.float32)
            for l in range(D // 16):
                r0[i, pl.ds(l * 16, 16)] = zero
            return carry

        lax.fori_loop(0, K, fill_body, 0)
        for off, sz in _PIECES:
            pltpu.sync_copy(r0.at[pl.ds(0, sz)],
                            acc.at[pl.ds(rowbase + off, sz)])
        idx_wait(0, 0)
        pltpu.async_copy(feat.at[sidx[0]], r0, sem0)
        idx_wait(1, 1)
        pltpu.async_copy(feat.at[sidx[1]], r1, sem1)
        plsc.subcore_barrier()

        def step(jc, b, p, prefetch_idx, start_next):
            pltpu.make_async_copy(feat.at[sidx[p]], rows[b], gsem[b]).wait()
            pltpu.sync_copy(rows[b], acc.at[didx[p]], add=True)
            if prefetch_idx:
                idx_start(jc + 4, p)
            if start_next:
                pn = (p + 2) % 4
                idx_wait(jc + 2, pn)
                pltpu.async_copy(feat.at[sidx[pn]], rows[b], gsem[b])

        def loop_body(j4, carry):
            for b4 in range(4):
                step(4 * j4 + b4, b4 % 2, b4, True, True)
            return carry

        lax.fori_loop(0, (C - 4) // 4, loop_body, 0)
        for b4 in range(4):
            step(C - 4 + b4, b4 % 2, b4, False, b4 < 2)

        plsc.subcore_barrier()
        for off, sz in _PIECES:
            pltpu.sync_copy(acc.at[pl.ds(rowbase + off, sz)],
                            r0.at[pl.ds(0, sz)])
            pltpu.sync_copy(r0.at[pl.ds(0, sz)],
                            out.at[c, pl.ds(rowbase + off, sz)])

        if not with_deg:
            return

        def fill2_body(i, carry):
            zero = jnp.zeros((16,), jnp.float32)
            one = jnp.full((16,), 1.0, dtype=jnp.float32)
            for l in range(D // 16):
                r0[i, pl.ds(l * 16, 16)] = zero
                r1[i, pl.ds(l * 16, 16)] = one
            return carry

        lax.fori_loop(0, K, fill2_body, 0)

        def didx_start(jc, p):
            pltpu.async_copy(dstc.at[pl.ds(base + jc * K, K)], didx[p],
                             isem[p])

        def didx_wait(jc, p):
            pltpu.make_async_copy(dstc.at[pl.ds(base + jc * K, K)], didx[p],
                                  isem[p]).wait()

        didx_start(0, 0)
        didx_start(1, 1)
        for off, sz in _PIECES:
            pltpu.sync_copy(r0.at[pl.ds(0, sz)],
                            acc.at[pl.ds(rowbase + off, sz)])
        plsc.subcore_barrier()

        def dstep(jc, p, prefetch_idx):
            didx_wait(jc, p)
            pltpu.sync_copy(r1, acc.at[didx[p]], add=True)
            if prefetch_idx:
                didx_start(jc + 2, p)

        def dloop_body(j2, carry):
            for b2 in range(2):
                dstep(2 * j2 + b2, b2, True)
            return carry

        lax.fori_loop(0, (C - 2) // 2, dloop_body, 0)
        dstep(C - 2, 0, False)
        dstep(C - 1, 1, False)

        plsc.subcore_barrier()
        for off, sz in _PIECES:
            pltpu.sync_copy(acc.at[pl.ds(rowbase + off, sz)],
                            r0.at[pl.ds(0, sz)])
            pltpu.sync_copy(r0.at[pl.ds(0, sz)],
                            outd.at[c, pl.ds(rowbase + off, sz)])

    out_type = jax.ShapeDtypeStruct((NC, NACC, D), jnp.float32)
    return pl.kernel(
        body,
        out_type=(out_type, out_type) if with_deg else out_type,
        mesh=plsc.VectorSubcoreMesh(**_MESH), scratch_types=scratch,
        name="sage_agg_deg" if with_deg else "sage_agg")


_R = 2000


def _make_lin_kernel(relu: bool):
    def body(p0, p1, d0, d1, xb, wl, wr, bb, ob):
        deg = jnp.maximum(d0[0, :, 0:1] + d1[0, :, 0:1], 1.0)
        mean = (p0[0] + p1[0]) / deg
        acc = jnp.dot(mean, wl[...], preferred_element_type=jnp.float32)
        acc = acc + jnp.dot(xb[...], wr[...],
                            preferred_element_type=jnp.float32)
        acc = acc + bb[...]
        if relu:
            acc = jnp.maximum(acc, 0.0)
        ob[...] = acc

    c0 = lambda i: (0, i, 0)
    c1 = lambda i: (1, i, 0)
    row = lambda i: (i, 0)
    full = lambda i: (0, 0)
    return pl.pallas_call(
        body,
        grid=(N // _R,),
        in_specs=[
            pl.BlockSpec((1, _R, D), c0),
            pl.BlockSpec((1, _R, D), c1),
            pl.BlockSpec((1, _R, D), c0),
            pl.BlockSpec((1, _R, D), c1),
            pl.BlockSpec((_R, D), row),
            pl.BlockSpec((D, D), full),
            pl.BlockSpec((D, D), full),
            pl.BlockSpec((1, D), full),
        ],
        out_specs=pl.BlockSpec((_R, D), row),
        out_shape=jax.ShapeDtypeStruct((N, D), jnp.float32),
        name="sage_lin_relu" if relu else "sage_lin",
    )


_agg = _make_agg_kernel()
_agg_deg = _make_agg_kernel(with_deg=True)
_lin_relu = _make_lin_kernel(relu=True)
_lin = _make_lin_kernel(relu=False)


def kernel(x, edge_index, W_l1, W_r1, b1, W_l2, W_r2, b2):
    src = edge_index[0].astype(jnp.int32)
    dst = edge_index[1].astype(jnp.int32)
    npd = EPAD - E
    pad_src = N + (jnp.arange(npd, dtype=jnp.int32) % 16)
    pad_dst = N + (jnp.arange(npd, dtype=jnp.int32) % (NACC - N))
    srcc = jnp.concatenate([src, pad_src])
    dstc = jnp.concatenate([dst, pad_dst])

    x_pad = jnp.pad(x, ((0, NPAD - N), (0, 0)))

    part1, degp = _agg_deg(x_pad, srcc, dstc)
    h = _lin_relu(part1, part1, degp, degp, x,
                  W_l1, W_r1, b1.reshape(1, D))
    h_pad = jnp.pad(h, ((0, NPAD - N), (0, 0)))
    part2 = _agg(h_pad, srcc, dstc)
    out = _lin(part2, part2, degp, degp, h,
               W_l2, W_r2, b2.reshape(1, D))
    return out


# speedup vs baseline: 1.1530x; 1.0231x over previous
import jax
import jax.numpy as jnp
from jax import lax
from jax.experimental import pallas as pl
from jax.experimental.pallas import tpu as pltpu
from jax.experimental.pallas import tpu_sc as plsc

N = 10000
D = 128
E = 320000
NPAD = N + 16
NC = 2
NS = 16
NW = NC * NS
K = 120
C = 84
EPAD = NW * C * K
NACC = 10112
RPT = NACC // NS

_PIECES = [(k * K, K) for k in range(RPT // K)]
if RPT % K:
    _PIECES.append((RPT - RPT % K, RPT % K))

_MESH = dict(core_axis_name="c", subcore_axis_name="s",
             num_cores=NC, num_subcores=NS)


def _worker():
    c = lax.axis_index("c")
    s = lax.axis_index("s")
    wid = c * NS + s
    return c, s * RPT, wid * C * K


def _make_agg_kernel(with_deg: bool = False):
    scratch = [
        pltpu.VMEM_SHARED((NACC, D), jnp.float32),
        pltpu.VMEM((K, D), jnp.float32),
        pltpu.VMEM((K, D), jnp.float32),
        pltpu.SemaphoreType.DMA,
        pltpu.SemaphoreType.DMA,
    ] + [pltpu.VMEM((K,), jnp.int32) for _ in range(8)] \
      + [pltpu.SemaphoreType.DMA for _ in range(4)]

    def body(feat, srcc, dstc, *rest):
        if with_deg:
            (out, outd, acc, r0, r1, sem0, sem1,
             si0, di0, si1, di1, si2, di2, si3, di3,
             smi0, smi1, smi2, smi3) = rest
        else:
            (out, acc, r0, r1, sem0, sem1,
             si0, di0, si1, di1, si2, di2, si3, di3,
             smi0, smi1, smi2, smi3) = rest
            outd = None
        c, rowbase, base = _worker()
        rows = (r0, r1)
        gsem = (sem0, sem1)
        sidx = (si0, si1, si2, si3)
        didx = (di0, di1, di2, di3)
        isem = (smi0, smi1, smi2, smi3)

        def idx_start(jc, p):
            pltpu.async_copy(srcc.at[pl.ds(base + jc * K, K)], sidx[p],
                             isem[p])
            pltpu.async_copy(dstc.at[pl.ds(base + jc * K, K)], didx[p],
                             isem[p])

        def idx_wait(jc, p):
            pltpu.make_async_copy(srcc.at[pl.ds(base + jc * K, K)], sidx[p],
                                  isem[p]).wait()
            pltpu.make_async_copy(dstc.at[pl.ds(base + jc * K, K)], didx[p],
                                  isem[p]).wait()

        for p in range(4):
            idx_start(p, p)

        def fill_body(i, carry):
            zero = jnp.zeros((16,), jnp.float32)
            for l in range(D // 16):
                r0[i, pl.ds(l * 16, 16)] = zero
            return carry

        lax.fori_loop(0, K, fill_body, 0)
        for off, sz in _PIECES:
            pltpu.sync_copy(r0.at[pl.ds(0, sz)],
                            acc.at[pl.ds(rowbase + off, sz)])
        idx_wait(0, 0)
        pltpu.async_copy(feat.at[sidx[0]], r0, sem0)
        idx_wait(1, 1)
        pltpu.async_copy(feat.at[sidx[1]], r1, sem1)
        plsc.subcore_barrier()

        def step(jc, b, p, prefetch_idx, start_next):
            pltpu.make_async_copy(feat.at[sidx[p]], rows[b], gsem[b]).wait()
            pltpu.sync_copy(rows[b], acc.at[didx[p]], add=True)
            if prefetch_idx:
                idx_start(jc + 4, p)
            if start_next:
                pn = (p + 2) % 4
                idx_wait(jc + 2, pn)
                pltpu.async_copy(feat.at[sidx[pn]], rows[b], gsem[b])

        def loop_body(j4, carry):
            for b4 in range(4):
                step(4 * j4 + b4, b4 % 2, b4, True, True)
            return carry

        lax.fori_loop(0, (C - 4) // 4, loop_body, 0)
        for b4 in range(4):
            step(C - 4 + b4, b4 % 2, b4, False, b4 < 2)

        plsc.subcore_barrier()
        for off, sz in _PIECES:
            pltpu.sync_copy(acc.at[pl.ds(rowbase + off, sz)],
                            r0.at[pl.ds(0, sz)])
            pltpu.sync_copy(r0.at[pl.ds(0, sz)],
                            out.at[c, pl.ds(rowbase + off, sz)])

        if not with_deg:
            return

        def fill2_body(i, carry):
            zero = jnp.zeros((16,), jnp.float32)
            one = jnp.full((16,), 1.0, dtype=jnp.float32)
            for l in range(D // 16):
                r0[i, pl.ds(l * 16, 16)] = zero
                r1[i, pl.ds(l * 16, 16)] = one
            return carry

        lax.fori_loop(0, K, fill2_body, 0)

        def didx_start(jc, p):
            pltpu.async_copy(dstc.at[pl.ds(base + jc * K, K)], didx[p],
                             isem[p])

        def didx_wait(jc, p):
            pltpu.make_async_copy(dstc.at[pl.ds(base + jc * K, K)], didx[p],
                                  isem[p]).wait()

        didx_start(0, 0)
        didx_start(1, 1)
        for off, sz in _PIECES:
            pltpu.sync_copy(r0.at[pl.ds(0, sz)],
                            acc.at[pl.ds(rowbase + off, sz)])
        plsc.subcore_barrier()

        def dstep(jc, p, prefetch_idx):
            didx_wait(jc, p)
            pltpu.sync_copy(r1, acc.at[didx[p]], add=True)
            if prefetch_idx:
                didx_start(jc + 2, p)

        def dloop_body(j2, carry):
            for b2 in range(2):
                dstep(2 * j2 + b2, b2, True)
            return carry

        lax.fori_loop(0, (C - 2) // 2, dloop_body, 0)
        dstep(C - 2, 0, False)
        dstep(C - 1, 1, False)

        plsc.subcore_barrier()
        for off, sz in _PIECES:
            pltpu.sync_copy(acc.at[pl.ds(rowbase + off, sz)],
                            r0.at[pl.ds(0, sz)])
            pltpu.sync_copy(r0.at[pl.ds(0, sz)],
                            outd.at[c, pl.ds(rowbase + off, sz)])

    out_type = jax.ShapeDtypeStruct((NC, NACC, D), jnp.float32)
    return pl.kernel(
        body,
        out_type=(out_type, out_type) if with_deg else out_type,
        mesh=plsc.VectorSubcoreMesh(**_MESH), scratch_types=scratch,
        name="sage_agg_deg" if with_deg else "sage_agg")


_R = 2000


def _make_lin_kernel(relu: bool):
    def body(p0, p1, d0, d1, xb, wl, wr, bb, ob):
        deg = jnp.maximum(d0[0, :, 0:1] + d1[0, :, 0:1], 1.0)
        mean = (p0[0] + p1[0]) / deg
        acc = jnp.dot(mean, wl[...], preferred_element_type=jnp.float32)
        acc = acc + jnp.dot(xb[...], wr[...],
                            preferred_element_type=jnp.float32)
        acc = acc + bb[...]
        if relu:
            acc = jnp.maximum(acc, 0.0)
        ob[...] = acc

    c0 = lambda i: (0, i, 0)
    c1 = lambda i: (1, i, 0)
    row = lambda i: (i, 0)
    full = lambda i: (0, 0)
    return pl.pallas_call(
        body,
        grid=(N // _R,),
        in_specs=[
            pl.BlockSpec((1, _R, D), c0),
            pl.BlockSpec((1, _R, D), c1),
            pl.BlockSpec((1, _R, D), c0),
            pl.BlockSpec((1, _R, D), c1),
            pl.BlockSpec((_R, D), row),
            pl.BlockSpec((D, D), full),
            pl.BlockSpec((D, D), full),
            pl.BlockSpec((1, D), full),
        ],
        out_specs=pl.BlockSpec((_R, D), row),
        out_shape=jax.ShapeDtypeStruct((N, D), jnp.float32),
        name="sage_lin_relu" if relu else "sage_lin",
    )


_agg = _make_agg_kernel()
_agg_deg = _make_agg_kernel(with_deg=True)
_lin_relu = _make_lin_kernel(relu=True)
_lin = _make_lin_kernel(relu=False)


def kernel(x, edge_index, W_l1, W_r1, b1, W_l2, W_r2, b2):
    src = edge_index[0].astype(jnp.int32)
    dst = edge_index[1].astype(jnp.int32)
    npd = EPAD - E
    pad_src = jnp.arange(npd, dtype=jnp.int32) % 4096
    pad_dst = N + (jnp.arange(npd, dtype=jnp.int32) % (NACC - N))
    srcc = jnp.concatenate([src, pad_src])
    dstc = jnp.concatenate([dst, pad_dst])

    part1, degp = _agg_deg(x, srcc, dstc)
    h = _lin_relu(part1, part1, degp, degp, x,
                  W_l1, W_r1, b1.reshape(1, D))
    part2 = _agg(h, srcc, dstc)
    out = _lin(part2, part2, degp, degp, h,
               W_l2, W_r2, b2.reshape(1, D))
    return out
